# baseline (speedup 1.0000x reference)
import functools

import jax
import jax.numpy as jnp
from jax import lax
from jax.experimental import pallas as pl
from jax.experimental.pallas import tpu as pltpu
from jax.experimental.pallas import tpu_sc as plsc

N_NODES = 10000
D_IN = 128
HEADS = 8
HID = 32
EMB = 128

NC = 2
NS = 16
LANES = 16
TW = 144
AW = 136
EBLK = 64

_f32 = jnp.float32
_i32 = jnp.int32


def _phase_a_body(x_ref, ws1_ref, wd1_ref, asa_ref, asb_ref, ad16_ref,
                  t1a_ref, t1b_ref, ad1t_ref):
    xb = x_ref[...]
    hs = jnp.dot(xb, ws1_ref[...], preferred_element_type=_f32)
    hd = jnp.dot(xb, wd1_ref[...], preferred_element_type=_f32)
    t1a_ref[...] = jnp.concatenate(
        [hs[:, :128], jnp.dot(hs, asa_ref[...], preferred_element_type=_f32)],
        axis=1)
    t1b_ref[...] = jnp.concatenate(
        [hs[:, 128:], jnp.dot(hs, asb_ref[...], preferred_element_type=_f32)],
        axis=1)
    ad1t_ref[...] = jnp.dot(hd, ad16_ref[...], preferred_element_type=_f32)


def _phase_a(x, ws1, wd1, asa, asb, ad16):
    n = x.shape[0]
    blk = 1000
    grid = n // blk
    full = lambda shape: pl.BlockSpec(shape, lambda i: (0, 0))
    return pl.pallas_call(
        _phase_a_body,
        grid=(grid,),
        in_specs=[
            pl.BlockSpec((blk, D_IN), lambda i: (i, 0)),
            full((D_IN, HEADS * HID)),
            full((D_IN, HEADS * HID)),
            full((HEADS * HID, 16)),
            full((HEADS * HID, 16)),
            full((HEADS * HID, 16)),
        ],
        out_specs=[
            pl.BlockSpec((blk, TW), lambda i: (i, 0)),
            pl.BlockSpec((blk, TW), lambda i: (i, 0)),
            pl.BlockSpec((blk, 16), lambda i: (i, 0)),
        ],
        out_shape=[
            jax.ShapeDtypeStruct((n, TW), _f32),
            jax.ShapeDtypeStruct((n, TW), _f32),
            jax.ShapeDtypeStruct((n, 16), _f32),
        ],
    )(x, ws1, wd1, asa, asb, ad16)


def _phase_c_body(acca_ref, accb_ref, b1_ref, ws2_ref, wd2_ref, as2p_ref,
                  ad2p_ref, r8_ref, t2_ref, ad2t_ref):
    acca = acca_ref[...]
    accb = accb_ref[...]
    den8 = jnp.concatenate([acca[:, 128:132], accb[:, 128:132]], axis=1)
    rec8 = 1.0 / (den8 + 1e-16)
    scale = jnp.dot(rec8, r8_ref[...], preferred_element_type=_f32)
    num = jnp.concatenate([acca[:, :128], accb[:, :128]], axis=1)
    h1 = jnp.maximum(num * scale + b1_ref[...], 0.0)
    h2s = jnp.dot(h1, ws2_ref[...], preferred_element_type=_f32)
    hd2 = jnp.dot(h1, wd2_ref[...], preferred_element_type=_f32)
    t2_ref[...] = jnp.concatenate(
        [h2s, jnp.dot(h2s, as2p_ref[...], preferred_element_type=_f32)],
        axis=1)
    ad2t_ref[...] = jnp.dot(hd2, ad2p_ref[...], preferred_element_type=_f32)


def _phase_c(acca, accb, b1row, ws2, wd2, as2p, ad2p, r8):
    n = acca.shape[0]
    blk = 1000
    grid = n // blk
    full = lambda shape: pl.BlockSpec(shape, lambda i: (0, 0))
    d2 = HEADS * HID
    return pl.pallas_call(
        _phase_c_body,
        grid=(grid,),
        in_specs=[
            pl.BlockSpec((blk, AW), lambda i: (i, 0)),
            pl.BlockSpec((blk, AW), lambda i: (i, 0)),
            full((1, d2)),
            full((d2, EMB)),
            full((d2, EMB)),
            full((EMB, 16)),
            full((EMB, 16)),
            full((HEADS, d2)),
        ],
        out_specs=[
            pl.BlockSpec((blk, TW), lambda i: (i, 0)),
            pl.BlockSpec((blk, 16), lambda i: (i, 0)),
        ],
        out_shape=[
            jax.ShapeDtypeStruct((n, TW), _f32),
            jax.ShapeDtypeStruct((n, 16), _f32),
        ],
    )(acca, accb, b1row, ws2, wd2, as2p, ad2p, r8)


def _phase_e_body(acca_ref, accb_ref, b2_ref, out_ref):
    acca = acca_ref[...]
    accb = accb_ref[...]
    num = acca[:, :EMB] + accb[:, :EMB]
    den = acca[:, 128:129] + accb[:, 128:129]
    out_ref[...] = num / (den + 1e-16) + b2_ref[...]


def _phase_e(acca, accb, b2row):
    n = acca.shape[0]
    blk = 1000
    grid = n // blk
    return pl.pallas_call(
        _phase_e_body,
        grid=(grid,),
        in_specs=[
            pl.BlockSpec((blk, AW), lambda i: (i, 0)),
            pl.BlockSpec((blk, AW), lambda i: (i, 0)),
            pl.BlockSpec((1, EMB), lambda i: (0, 0)),
        ],
        out_specs=pl.BlockSpec((blk, EMB), lambda i: (i, 0)),
        out_shape=jax.ShapeDtypeStruct((n, EMB), _f32),
    )(acca, accb, b2row)


def _sc_edge_layer(tab, adt, epk, zrs, n, nheads, split_edges):
    e = epk.shape[0] * EBLK
    eb = e // EBLK
    share = eb // NC if split_edges else eb
    rpt = 8 * (n // (8 * NS))
    rem = n - NS * rpt

    mesh = plsc.VectorSubcoreMesh(core_axis_name="c", subcore_axis_name="s")

    def body(tab_hbm, adt_hbm, epk_hbm, zrs_hbm,
             oa_hbm, ob_hbm,
             acc, ib0, ib1, ib2, ib3, rows0, rows1, ad0, ad1m, msg0, msg1,
             isem0, isem1, isem2, isem3,
             grsem0, grsem1, gasem0, gasem1, ssem0, ssem1):
        ibs = [ib0, ib1, ib2, ib3]
        isems = [isem0, isem1, isem2, isem3]
        rowsb = [rows0, rows1]
        adb = [ad0, ad1m]
        msgb = [msg0, msg1]
        grs = [grsem0, grsem1]
        gas = [gasem0, gasem1]
        sss = [ssem0, ssem1]

        c = lax.axis_index("c")
        s = lax.axis_index("s")

        pltpu.sync_copy(zrs_hbm.at[pl.ds(s * rpt, rpt)],
                        acc.at[pl.ds(s * rpt, rpt)])
        if rem:
            @pl.when(s == NS - 1)
            def _():
                pltpu.sync_copy(zrs_hbm.at[pl.ds(NS * rpt, rem)],
                                acc.at[pl.ds(NS * rpt, rem)])

        def zpad(i, _):
            msg0[i, pl.ds(AW - 16, 16)] = jnp.zeros((16,), _f32)
            msg1[i, pl.ds(AW - 16, 16)] = jnp.zeros((16,), _f32)
            return 0
        lax.fori_loop(0, EBLK, zpad, 0)
        plsc.subcore_barrier()

        iota16 = lax.iota(_i32, LANES)

        gbase = (c * share) if split_edges else 0
        nblk = (share - s + NS - 1) // NS
        ad_off = 0 if split_edges else c * nheads
        ch = 128 // nheads

        def gblk(k):
            return gbase + s + k * NS

        def adjust_src(ib):
            if not split_edges:
                coff = jnp.full((LANES,), c * n, _i32)
                for m in range(EBLK // LANES):
                    ib[0, pl.ds(m * LANES, LANES)] = (
                        ib[0, pl.ds(m * LANES, LANES)] + coff)

        def compute_block(rows, adrows, msg):
            def group(i, _):
                e16 = i * LANES + iota16
                for h in range(nheads):
                    asv = plsc.load_gather(
                        rows, [e16, jnp.full((LANES,), 128 + h, _i32)])
                    adv = plsc.load_gather(
                        adrows, [e16, jnp.full((LANES,), ad_off + h, _i32)])
                    al = asv + adv
                    ex = jnp.exp(jnp.maximum(al, 0.2 * al))
                    plsc.store_scatter(
                        msg, [e16, jnp.full((LANES,), 128 + h, _i32)], ex)
                    for j in range(LANES):
                        exv = jnp.full((LANES,), ex[j], _f32)
                        ei = i * LANES + j
                        for q in range(ch // LANES):
                            col = h * ch + q * LANES
                            msg[ei, pl.ds(col, LANES)] = (
                                rows[ei, pl.ds(col, LANES)] * exv)
                return 0
            lax.fori_loop(0, EBLK // LANES, group, 0)

        def start_gathers(u):
            pltpu.async_copy(tab_hbm.at[ibs[u % 4].at[0]],
                             rowsb[u % 2], grs[u % 2])
            pltpu.async_copy(adt_hbm.at[ibs[u % 4].at[1]],
                             adb[u % 2], gas[u % 2])

        def wait_gathers(u):
            pltpu.make_async_copy(tab_hbm.at[ibs[u % 4].at[0]],
                                  rowsb[u % 2], grs[u % 2]).wait()
            pltpu.make_async_copy(adt_hbm.at[ibs[u % 4].at[1]],
                                  adb[u % 2], gas[u % 2]).wait()

        pltpu.sync_copy(epk_hbm.at[gblk(0)], ib0)
        adjust_src(ib0)
        start_gathers(0)

        @pl.when(1 < nblk)
        def _():
            pltpu.async_copy(epk_hbm.at[gblk(1)], ib1, isem1)

        def quad(kk, _):
            for u in range(4):
                k = kk * 4 + u

                @pl.when(k + 1 < nblk)
                def _(u=u, k=k):
                    pltpu.make_async_copy(
                        epk_hbm.at[gblk(k + 1)],
                        ibs[(u + 1) % 4], isems[(u + 1) % 4]).wait()
                    adjust_src(ibs[(u + 1) % 4])
                    start_gathers(u + 1)

                @pl.when(jnp.logical_and(k >= 2, k <= nblk + 1))
                def _(u=u, k=k):
                    pltpu.make_async_copy(
                        msgb[u % 2], acc.at[ibs[(u + 2) % 4].at[1]],
                        sss[u % 2]).wait()

                @pl.when(k + 2 < nblk)
                def _(u=u, k=k):
                    pltpu.async_copy(epk_hbm.at[gblk(k + 2)],
                                     ibs[(u + 2) % 4], isems[(u + 2) % 4])

                @pl.when(k < nblk)
                def _(u=u, k=k):
                    wait_gathers(u)
                    compute_block(rowsb[u % 2], adb[u % 2], msgb[u % 2])
                    pltpu.async_copy(msgb[u % 2], acc.at[ibs[u % 4].at[1]],
                                     sss[u % 2], add=True)
            return 0

        lax.fori_loop(0, (nblk + 2 + 3) // 4, quad, 0)
        plsc.subcore_barrier()

        def drain(o_hbm):
            pltpu.sync_copy(acc.at[pl.ds(s * rpt, rpt)],
                            o_hbm.at[pl.ds(s * rpt, rpt)])
            if rem:
                @pl.when(s == NS - 1)
                def _():
                    pltpu.sync_copy(acc.at[pl.ds(NS * rpt, rem)],
                                    o_hbm.at[pl.ds(NS * rpt, rem)])

        @pl.when(c == 0)
        def _():
            drain(oa_hbm)

        @pl.when(c == 1)
        def _():
            drain(ob_hbm)

    kern = pl.kernel(
        body,
        out_type=[
            jax.ShapeDtypeStruct((n, AW), _f32),
            jax.ShapeDtypeStruct((n, AW), _f32),
        ],
        mesh=mesh,
        scratch_types=(
            [pltpu.VMEM_SHARED((n, AW), _f32)]
            + [pltpu.VMEM((2, EBLK), _i32) for _ in range(4)]
            + [pltpu.VMEM((EBLK, TW), _f32) for _ in range(2)]
            + [pltpu.VMEM((EBLK, 16), _f32) for _ in range(2)]
            + [pltpu.VMEM((EBLK, AW), _f32) for _ in range(2)]
            + [pltpu.SemaphoreType.DMA for _ in range(10)]
        ),
        compiler_params=pltpu.CompilerParams(use_tc_tiling_on_sc=False,
                                             needs_layout_passes=False),
    )
    return kern(tab, adt, epk, zrs)


def kernel(x, edge_index, Ws1, Wd1, as1, ad1, b1, Ws2, Wd2, as2, ad2, b2):
    n = x.shape[0]
    ei = edge_index.astype(_i32)
    src = ei[0]
    dst = ei[1]

    m1 = jnp.repeat(jnp.eye(HEADS, dtype=_f32), HID, axis=0)
    as8 = m1 * as1.reshape(-1)[:, None]
    ad8 = m1 * ad1.reshape(-1)[:, None]
    pad12 = jnp.zeros((HEADS * HID, 12), _f32)
    pad8 = jnp.zeros((HEADS * HID, 8), _f32)
    asa = jnp.concatenate([as8[:, :4], pad12], axis=1)
    asb = jnp.concatenate([as8[:, 4:], pad12], axis=1)
    ad16 = jnp.concatenate([ad8, pad8], axis=1)
    as2p = jnp.concatenate([as2.T, jnp.zeros((EMB, 15), _f32)], axis=1)
    ad2p = jnp.concatenate([ad2.T, jnp.zeros((EMB, 15), _f32)], axis=1)
    r8 = jnp.repeat(jnp.eye(HEADS, dtype=_f32), HID, axis=1)
    b1row = b1.reshape(1, -1)
    b2row = b2.reshape(1, -1)
    zrs = jnp.zeros((n, AW), _f32)
    eb = src.shape[0] // EBLK
    epk = jnp.stack([src.reshape(eb, EBLK), dst.reshape(eb, EBLK)], axis=1)

    t1a, t1b, ad1t = _phase_a(x, Ws1, Wd1, asa, asb, ad16)
    tab1 = jnp.concatenate([t1a, t1b], axis=0)
    acc1a, acc1b = _sc_edge_layer(tab1, ad1t, epk, zrs, n,
                                  nheads=4, split_edges=False)
    t2, ad2t = _phase_c(acc1a, acc1b, b1row, Ws2, Wd2, as2p, ad2p, r8)
    acc2a, acc2b = _sc_edge_layer(t2, ad2t, epk, zrs, n,
                                  nheads=1, split_edges=True)
    return _phase_e(acc2a, acc2b, b2row)


# speedup vs baseline: 2.1200x; 2.1200x over previous
import functools

import jax
import jax.numpy as jnp
from jax import lax
from jax.experimental import pallas as pl
from jax.experimental.pallas import tpu as pltpu
from jax.experimental.pallas import tpu_sc as plsc

N_NODES = 10000
D_IN = 128
HEADS = 8
HID = 32
EMB = 128

NC = 2
NS = 16
LANES = 16
TW = 144
AW = 136
EBLK = 64

_f32 = jnp.float32
_i32 = jnp.int32


def _phase_a_body(x_ref, ws1_ref, wd1_ref, asa_ref, asb_ref, ad16_ref,
                  t1a_ref, t1b_ref, ad1t_ref):
    xb = x_ref[...]
    hs = jnp.dot(xb, ws1_ref[...], preferred_element_type=_f32)
    hd = jnp.dot(xb, wd1_ref[...], preferred_element_type=_f32)
    t1a_ref[...] = jnp.concatenate(
        [hs[:, :128], jnp.dot(hs, asa_ref[...], preferred_element_type=_f32)],
        axis=1)
    t1b_ref[...] = jnp.concatenate(
        [hs[:, 128:], jnp.dot(hs, asb_ref[...], preferred_element_type=_f32)],
        axis=1)
    ad1t_ref[...] = jnp.dot(hd, ad16_ref[...], preferred_element_type=_f32)


def _phase_a(x, ws1, wd1, asa, asb, ad16):
    n = x.shape[0]
    blk = 1000
    grid = n // blk
    full = lambda shape: pl.BlockSpec(shape, lambda i: (0, 0))
    return pl.pallas_call(
        _phase_a_body,
        grid=(grid,),
        in_specs=[
            pl.BlockSpec((blk, D_IN), lambda i: (i, 0)),
            full((D_IN, HEADS * HID)),
            full((D_IN, HEADS * HID)),
            full((HEADS * HID, 16)),
            full((HEADS * HID, 16)),
            full((HEADS * HID, 16)),
        ],
        out_specs=[
            pl.BlockSpec((blk, TW), lambda i: (i, 0)),
            pl.BlockSpec((blk, TW), lambda i: (i, 0)),
            pl.BlockSpec((blk, 16), lambda i: (i, 0)),
        ],
        out_shape=[
            jax.ShapeDtypeStruct((n, TW), _f32),
            jax.ShapeDtypeStruct((n, TW), _f32),
            jax.ShapeDtypeStruct((n, 16), _f32),
        ],
    )(x, ws1, wd1, asa, asb, ad16)


def _phase_c_body(acca_ref, accb_ref, b1_ref, ws2_ref, wd2_ref, as2p_ref,
                  ad2p_ref, r8_ref, t2_ref, ad2t_ref):
    acca = acca_ref[...]
    accb = accb_ref[...]
    den8 = jnp.concatenate([acca[:, 128:132], accb[:, 128:132]], axis=1)
    rec8 = 1.0 / (den8 + 1e-16)
    scale = jnp.dot(rec8, r8_ref[...], preferred_element_type=_f32)
    num = jnp.concatenate([acca[:, :128], accb[:, :128]], axis=1)
    h1 = jnp.maximum(num * scale + b1_ref[...], 0.0)
    h2s = jnp.dot(h1, ws2_ref[...], preferred_element_type=_f32)
    hd2 = jnp.dot(h1, wd2_ref[...], preferred_element_type=_f32)
    t2_ref[...] = jnp.concatenate(
        [h2s, jnp.dot(h2s, as2p_ref[...], preferred_element_type=_f32)],
        axis=1)
    ad2t_ref[...] = jnp.dot(hd2, ad2p_ref[...], preferred_element_type=_f32)


def _phase_c(acca, accb, b1row, ws2, wd2, as2p, ad2p, r8):
    n = acca.shape[0]
    blk = 1000
    grid = n // blk
    full = lambda shape: pl.BlockSpec(shape, lambda i: (0, 0))
    d2 = HEADS * HID
    return pl.pallas_call(
        _phase_c_body,
        grid=(grid,),
        in_specs=[
            pl.BlockSpec((blk, AW), lambda i: (i, 0)),
            pl.BlockSpec((blk, AW), lambda i: (i, 0)),
            full((1, d2)),
            full((d2, EMB)),
            full((d2, EMB)),
            full((EMB, 16)),
            full((EMB, 16)),
            full((HEADS, d2)),
        ],
        out_specs=[
            pl.BlockSpec((blk, TW), lambda i: (i, 0)),
            pl.BlockSpec((blk, 16), lambda i: (i, 0)),
        ],
        out_shape=[
            jax.ShapeDtypeStruct((n, TW), _f32),
            jax.ShapeDtypeStruct((n, 16), _f32),
        ],
    )(acca, accb, b1row, ws2, wd2, as2p, ad2p, r8)


def _phase_e_body(acca_ref, accb_ref, b2_ref, out_ref):
    acca = acca_ref[...]
    accb = accb_ref[...]
    num = acca[:, :EMB] + accb[:, :EMB]
    den = acca[:, 128:129] + accb[:, 128:129]
    out_ref[...] = num / (den + 1e-16) + b2_ref[...]


def _phase_e(acca, accb, b2row):
    n = acca.shape[0]
    blk = 1000
    grid = n // blk
    return pl.pallas_call(
        _phase_e_body,
        grid=(grid,),
        in_specs=[
            pl.BlockSpec((blk, AW), lambda i: (i, 0)),
            pl.BlockSpec((blk, AW), lambda i: (i, 0)),
            pl.BlockSpec((1, EMB), lambda i: (0, 0)),
        ],
        out_specs=pl.BlockSpec((blk, EMB), lambda i: (i, 0)),
        out_shape=jax.ShapeDtypeStruct((n, EMB), _f32),
    )(acca, accb, b2row)


def _sc_edge_layer(tab, adt, epk, zrs, n, nheads, split_edges):
    e = epk.shape[0] * EBLK
    eb = e // EBLK
    share = eb // NC if split_edges else eb
    rpt = 8 * (n // (8 * NS))
    rem = n - NS * rpt

    mesh = plsc.VectorSubcoreMesh(core_axis_name="c", subcore_axis_name="s")

    def body(tab_hbm, adt_hbm, epk_hbm, zrs_hbm,
             oa_hbm, ob_hbm,
             acc, ib0, ib1, ib2, ib3, rows0, rows1, ad0, ad1m, msg0, msg1,
             isem0, isem1, isem2, isem3,
             grsem0, grsem1, gasem0, gasem1, ssem0, ssem1):
        ibs = [ib0, ib1, ib2, ib3]
        isems = [isem0, isem1, isem2, isem3]
        rowsb = [rows0, rows1]
        adb = [ad0, ad1m]
        msgb = [msg0, msg1]
        grs = [grsem0, grsem1]
        gas = [gasem0, gasem1]
        sss = [ssem0, ssem1]

        c = lax.axis_index("c")
        s = lax.axis_index("s")

        pltpu.sync_copy(zrs_hbm.at[pl.ds(s * rpt, rpt)],
                        acc.at[pl.ds(s * rpt, rpt)])
        if rem:
            @pl.when(s == NS - 1)
            def _():
                pltpu.sync_copy(zrs_hbm.at[pl.ds(NS * rpt, rem)],
                                acc.at[pl.ds(NS * rpt, rem)])

        def zpad(i, _):
            msg0[i, pl.ds(AW - 16, 16)] = jnp.zeros((16,), _f32)
            msg1[i, pl.ds(AW - 16, 16)] = jnp.zeros((16,), _f32)
            return 0
        lax.fori_loop(0, EBLK, zpad, 0)
        plsc.subcore_barrier()

        iota16 = lax.iota(_i32, LANES)

        gbase = (c * share) if split_edges else 0
        nblk = (share - s + NS - 1) // NS
        ad_off = 0 if split_edges else c * nheads
        ch = 128 // nheads

        def gblk(k):
            return gbase + s + k * NS

        def adjust_src(ib):
            if not split_edges:
                coff = jnp.full((LANES,), c * n, _i32)
                for m in range(EBLK // LANES):
                    ib[0, pl.ds(m * LANES, LANES)] = (
                        ib[0, pl.ds(m * LANES, LANES)] + coff)

        nv = 128 // LANES

        def compute_block(rows, adrows, msg):
            def group(i, _):
                e16 = i * LANES + iota16
                exs = []
                for h in range(nheads):
                    asv = plsc.load_gather(
                        rows, [e16, jnp.full((LANES,), 128 + h, _i32)])
                    adv = plsc.load_gather(
                        adrows, [e16, jnp.full((LANES,), ad_off + h, _i32)])
                    al = asv + adv
                    ex = jnp.exp(jnp.maximum(al, 0.2 * al))
                    plsc.store_scatter(
                        msg, [e16, jnp.full((LANES,), 128 + h, _i32)], ex)
                    exs.append(ex)
                for j in range(LANES):
                    ei = i * LANES + j
                    loads = [rows[ei, pl.ds(v * LANES, LANES)]
                             for v in range(nv)]
                    vals = []
                    for h in range(nheads):
                        exv = jnp.full((LANES,), exs[h][j], _f32)
                        for q in range(ch // LANES):
                            vals.append(loads[h * (ch // LANES) + q] * exv)
                    for v in range(nv):
                        msg[ei, pl.ds(v * LANES, LANES)] = vals[v]
                return 0
            lax.fori_loop(0, EBLK // LANES, group, 0)

        def start_gathers(u):
            pltpu.async_copy(tab_hbm.at[ibs[u % 4].at[0]],
                             rowsb[u % 2], grs[u % 2])
            pltpu.async_copy(adt_hbm.at[ibs[u % 4].at[1]],
                             adb[u % 2], gas[u % 2])

        def wait_gathers(u):
            pltpu.make_async_copy(tab_hbm.at[ibs[u % 4].at[0]],
                                  rowsb[u % 2], grs[u % 2]).wait()
            pltpu.make_async_copy(adt_hbm.at[ibs[u % 4].at[1]],
                                  adb[u % 2], gas[u % 2]).wait()

        pltpu.sync_copy(epk_hbm.at[gblk(0)], ib0)
        adjust_src(ib0)
        start_gathers(0)

        @pl.when(1 < nblk)
        def _():
            pltpu.async_copy(epk_hbm.at[gblk(1)], ib1, isem1)

        def quad(kk, _):
            for u in range(4):
                k = kk * 4 + u

                @pl.when(k + 1 < nblk)
                def _(u=u, k=k):
                    pltpu.make_async_copy(
                        epk_hbm.at[gblk(k + 1)],
                        ibs[(u + 1) % 4], isems[(u + 1) % 4]).wait()
                    adjust_src(ibs[(u + 1) % 4])
                    start_gathers(u + 1)

                @pl.when(jnp.logical_and(k >= 2, k <= nblk + 1))
                def _(u=u, k=k):
                    pltpu.make_async_copy(
                        msgb[u % 2], acc.at[ibs[(u + 2) % 4].at[1]],
                        sss[u % 2]).wait()

                @pl.when(k + 2 < nblk)
                def _(u=u, k=k):
                    pltpu.async_copy(epk_hbm.at[gblk(k + 2)],
                                     ibs[(u + 2) % 4], isems[(u + 2) % 4])

                @pl.when(k < nblk)
                def _(u=u, k=k):
                    wait_gathers(u)
                    compute_block(rowsb[u % 2], adb[u % 2], msgb[u % 2])
                    pltpu.async_copy(msgb[u % 2], acc.at[ibs[u % 4].at[1]],
                                     sss[u % 2], add=True)
            return 0

        lax.fori_loop(0, (nblk + 2 + 3) // 4, quad, 0)
        plsc.subcore_barrier()

        def drain(o_hbm):
            pltpu.sync_copy(acc.at[pl.ds(s * rpt, rpt)],
                            o_hbm.at[pl.ds(s * rpt, rpt)])
            if rem:
                @pl.when(s == NS - 1)
                def _():
                    pltpu.sync_copy(acc.at[pl.ds(NS * rpt, rem)],
                                    o_hbm.at[pl.ds(NS * rpt, rem)])

        @pl.when(c == 0)
        def _():
            drain(oa_hbm)

        @pl.when(c == 1)
        def _():
            drain(ob_hbm)

    kern = pl.kernel(
        body,
        out_type=[
            jax.ShapeDtypeStruct((n, AW), _f32),
            jax.ShapeDtypeStruct((n, AW), _f32),
        ],
        mesh=mesh,
        scratch_types=(
            [pltpu.VMEM_SHARED((n, AW), _f32)]
            + [pltpu.VMEM((2, EBLK), _i32) for _ in range(4)]
            + [pltpu.VMEM((EBLK, TW), _f32) for _ in range(2)]
            + [pltpu.VMEM((EBLK, 16), _f32) for _ in range(2)]
            + [pltpu.VMEM((EBLK, AW), _f32) for _ in range(2)]
            + [pltpu.SemaphoreType.DMA for _ in range(10)]
        ),
        compiler_params=pltpu.CompilerParams(use_tc_tiling_on_sc=False,
                                             needs_layout_passes=False),
    )
    return kern(tab, adt, epk, zrs)


def kernel(x, edge_index, Ws1, Wd1, as1, ad1, b1, Ws2, Wd2, as2, ad2, b2):
    n = x.shape[0]
    ei = edge_index.astype(_i32)
    src = ei[0]
    dst = ei[1]

    m1 = jnp.repeat(jnp.eye(HEADS, dtype=_f32), HID, axis=0)
    as8 = m1 * as1.reshape(-1)[:, None]
    ad8 = m1 * ad1.reshape(-1)[:, None]
    pad12 = jnp.zeros((HEADS * HID, 12), _f32)
    pad8 = jnp.zeros((HEADS * HID, 8), _f32)
    asa = jnp.concatenate([as8[:, :4], pad12], axis=1)
    asb = jnp.concatenate([as8[:, 4:], pad12], axis=1)
    ad16 = jnp.concatenate([ad8, pad8], axis=1)
    as2p = jnp.concatenate([as2.T, jnp.zeros((EMB, 15), _f32)], axis=1)
    ad2p = jnp.concatenate([ad2.T, jnp.zeros((EMB, 15), _f32)], axis=1)
    r8 = jnp.repeat(jnp.eye(HEADS, dtype=_f32), HID, axis=1)
    b1row = b1.reshape(1, -1)
    b2row = b2.reshape(1, -1)
    zrs = jnp.zeros((n, AW), _f32)
    eb = src.shape[0] // EBLK
    epk = jnp.stack([src.reshape(eb, EBLK), dst.reshape(eb, EBLK)], axis=1)

    t1a, t1b, ad1t = _phase_a(x, Ws1, Wd1, asa, asb, ad16)
    tab1 = jnp.concatenate([t1a, t1b], axis=0)
    acc1a, acc1b = _sc_edge_layer(tab1, ad1t, epk, zrs, n,
                                  nheads=4, split_edges=False)
    t2, ad2t = _phase_c(acc1a, acc1b, b1row, Ws2, Wd2, as2p, ad2p, r8)
    acc2a, acc2b = _sc_edge_layer(t2, ad2t, epk, zrs, n,
                                  nheads=1, split_edges=True)
    return _phase_e(acc2a, acc2b, b2row)


# speedup vs baseline: 2.1972x; 1.0364x over previous
import functools

import jax
import jax.numpy as jnp
from jax import lax
from jax.experimental import pallas as pl
from jax.experimental.pallas import tpu as pltpu
from jax.experimental.pallas import tpu_sc as plsc

N_NODES = 10000
D_IN = 128
HEADS = 8
HID = 32
EMB = 128

NC = 2
NS = 16
LANES = 16
TW = 144
AW = 136
EBLK = 64

_f32 = jnp.float32
_i32 = jnp.int32


def _phase_a_body(x_ref, ws1_ref, wd1_ref, asa_ref, asb_ref, ad16_ref,
                  t1_ref, ad1t_ref):
    xb = x_ref[...]
    hs = jnp.dot(xb, ws1_ref[...], preferred_element_type=_f32)
    hd = jnp.dot(xb, wd1_ref[...], preferred_element_type=_f32)
    t1_ref[...] = jnp.concatenate(
        [hs[:, :128], jnp.dot(hs, asa_ref[...], preferred_element_type=_f32),
         hs[:, 128:], jnp.dot(hs, asb_ref[...], preferred_element_type=_f32)],
        axis=1)
    ad1t_ref[...] = jnp.dot(hd, ad16_ref[...], preferred_element_type=_f32)


def _phase_a(x, ws1, wd1, asa, asb, ad16):
    n = x.shape[0]
    blk = 1000
    grid = n // blk
    full = lambda shape: pl.BlockSpec(shape, lambda i: (0, 0))
    return pl.pallas_call(
        _phase_a_body,
        grid=(grid,),
        in_specs=[
            pl.BlockSpec((blk, D_IN), lambda i: (i, 0)),
            full((D_IN, HEADS * HID)),
            full((D_IN, HEADS * HID)),
            full((HEADS * HID, 16)),
            full((HEADS * HID, 16)),
            full((HEADS * HID, 16)),
        ],
        out_specs=[
            pl.BlockSpec((blk, 2 * TW), lambda i: (i, 0)),
            pl.BlockSpec((blk, 16), lambda i: (i, 0)),
        ],
        out_shape=[
            jax.ShapeDtypeStruct((n, 2 * TW), _f32),
            jax.ShapeDtypeStruct((n, 16), _f32),
        ],
    )(x, ws1, wd1, asa, asb, ad16)


def _phase_c_body(acca_ref, accb_ref, b1_ref, ws2_ref, wd2_ref, as2p_ref,
                  ad2p_ref, r8_ref, t2_ref, ad2t_ref):
    acca = acca_ref[...]
    accb = accb_ref[...]
    den8 = jnp.concatenate([acca[:, 128:132], accb[:, 128:132]], axis=1)
    rec8 = 1.0 / (den8 + 1e-16)
    scale = jnp.dot(rec8, r8_ref[...], preferred_element_type=_f32)
    num = jnp.concatenate([acca[:, :128], accb[:, :128]], axis=1)
    h1 = jnp.maximum(num * scale + b1_ref[...], 0.0)
    h2s = jnp.dot(h1, ws2_ref[...], preferred_element_type=_f32)
    hd2 = jnp.dot(h1, wd2_ref[...], preferred_element_type=_f32)
    t2_ref[...] = jnp.concatenate(
        [h2s, jnp.dot(h2s, as2p_ref[...], preferred_element_type=_f32)],
        axis=1)
    ad2t_ref[...] = jnp.dot(hd2, ad2p_ref[...], preferred_element_type=_f32)


def _phase_c(acca, accb, b1row, ws2, wd2, as2p, ad2p, r8):
    n = acca.shape[0]
    blk = 1000
    grid = n // blk
    full = lambda shape: pl.BlockSpec(shape, lambda i: (0, 0))
    d2 = HEADS * HID
    return pl.pallas_call(
        _phase_c_body,
        grid=(grid,),
        in_specs=[
            pl.BlockSpec((blk, AW), lambda i: (i, 0)),
            pl.BlockSpec((blk, AW), lambda i: (i, 0)),
            full((1, d2)),
            full((d2, EMB)),
            full((d2, EMB)),
            full((EMB, 16)),
            full((EMB, 16)),
            full((HEADS, d2)),
        ],
        out_specs=[
            pl.BlockSpec((blk, TW), lambda i: (i, 0)),
            pl.BlockSpec((blk, 16), lambda i: (i, 0)),
        ],
        out_shape=[
            jax.ShapeDtypeStruct((n, TW), _f32),
            jax.ShapeDtypeStruct((n, 16), _f32),
        ],
    )(acca, accb, b1row, ws2, wd2, as2p, ad2p, r8)


def _phase_e_body(acca_ref, accb_ref, b2_ref, out_ref):
    acca = acca_ref[...]
    accb = accb_ref[...]
    num = acca[:, :EMB] + accb[:, :EMB]
    den = acca[:, 128:129] + accb[:, 128:129]
    out_ref[...] = num / (den + 1e-16) + b2_ref[...]


def _phase_e(acca, accb, b2row):
    n = acca.shape[0]
    blk = 1000
    grid = n // blk
    return pl.pallas_call(
        _phase_e_body,
        grid=(grid,),
        in_specs=[
            pl.BlockSpec((blk, AW), lambda i: (i, 0)),
            pl.BlockSpec((blk, AW), lambda i: (i, 0)),
            pl.BlockSpec((1, EMB), lambda i: (0, 0)),
        ],
        out_specs=pl.BlockSpec((blk, EMB), lambda i: (i, 0)),
        out_shape=jax.ShapeDtypeStruct((n, EMB), _f32),
    )(acca, accb, b2row)


def _sc_edge_layer(tab, adt, epk, n, nheads, split_edges):
    e = epk.shape[0] * EBLK
    eb = e // EBLK
    share = eb // NC if split_edges else eb
    rpt = 8 * (n // (8 * NS))
    rem = n - NS * rpt

    mesh = plsc.VectorSubcoreMesh(core_axis_name="c", subcore_axis_name="s")

    def body(tab_hbm, adt_hbm, epk_hbm,
             oa_hbm, ob_hbm,
             acc, ib0, ib1, ib2, ib3, rows0, rows1, ad0, ad1m, msg0, msg1,
             isem0, isem1, isem2, isem3,
             grsem0, grsem1, gasem0, gasem1, ssem0, ssem1):
        ibs = [ib0, ib1, ib2, ib3]
        isems = [isem0, isem1, isem2, isem3]
        rowsb = [rows0, rows1]
        adb = [ad0, ad1m]
        msgb = [msg0, msg1]
        grs = [grsem0, grsem1]
        gas = [gasem0, gasem1]
        sss = [ssem0, ssem1]

        c = lax.axis_index("c")
        s = lax.axis_index("s")

        zcols = list(range(0, AW - 16, LANES)) + [AW - 16]

        def zfill(i, _):
            zv = jnp.zeros((LANES,), _f32)
            for col in zcols:
                msg0[i, pl.ds(col, LANES)] = zv
                msg1[i, pl.ds(col, LANES)] = zv
            return 0
        lax.fori_loop(0, EBLK, zfill, 0)

        nfull, tail = rpt // EBLK, rpt % EBLK

        def zacc(k, _):
            pltpu.sync_copy(msg0, acc.at[pl.ds(s * rpt + k * EBLK, EBLK)])
            return 0
        lax.fori_loop(0, nfull, zacc, 0)
        if tail:
            pltpu.sync_copy(msg0.at[pl.ds(0, tail)],
                            acc.at[pl.ds(s * rpt + nfull * EBLK, tail)])
        if rem:
            @pl.when(s == NS - 1)
            def _():
                pltpu.sync_copy(msg0.at[pl.ds(0, rem)],
                                acc.at[pl.ds(NS * rpt, rem)])
        plsc.subcore_barrier()

        iota16 = lax.iota(_i32, LANES)

        gbase = (c * share) if split_edges else 0
        nblk = (share - s + NS - 1) // NS
        ad_off = 0 if split_edges else c * nheads
        ch = 128 // nheads

        def gblk(k):
            return gbase + s + k * NS

        def adjust_src(ib):
            if not split_edges:
                coff = jnp.full((LANES,), c, _i32)
                for m in range(EBLK // LANES):
                    v = ib[0, pl.ds(m * LANES, LANES)]
                    ib[0, pl.ds(m * LANES, LANES)] = v + v + coff

        nv = 128 // LANES

        def compute_block(rows, adrows, msg):
            def group(i, _):
                e16 = i * LANES + iota16
                exs = []
                for h in range(nheads):
                    asv = plsc.load_gather(
                        rows, [e16, jnp.full((LANES,), 128 + h, _i32)])
                    adv = plsc.load_gather(
                        adrows, [e16, jnp.full((LANES,), ad_off + h, _i32)])
                    al = asv + adv
                    ex = jnp.exp(jnp.maximum(al, 0.2 * al))
                    plsc.store_scatter(
                        msg, [e16, jnp.full((LANES,), 128 + h, _i32)], ex)
                    exs.append(ex)
                for j in range(LANES):
                    ei = i * LANES + j
                    loads = [rows[ei, pl.ds(v * LANES, LANES)]
                             for v in range(nv)]
                    vals = []
                    for h in range(nheads):
                        exv = jnp.full((LANES,), exs[h][j], _f32)
                        for q in range(ch // LANES):
                            vals.append(loads[h * (ch // LANES) + q] * exv)
                    for v in range(nv):
                        msg[ei, pl.ds(v * LANES, LANES)] = vals[v]
                return 0
            lax.fori_loop(0, EBLK // LANES, group, 0)

        def start_gathers(u):
            pltpu.async_copy(tab_hbm.at[ibs[u % 4].at[0]],
                             rowsb[u % 2], grs[u % 2])
            pltpu.async_copy(adt_hbm.at[ibs[u % 4].at[1]],
                             adb[u % 2], gas[u % 2])

        def wait_gathers(u):
            pltpu.make_async_copy(tab_hbm.at[ibs[u % 4].at[0]],
                                  rowsb[u % 2], grs[u % 2]).wait()
            pltpu.make_async_copy(adt_hbm.at[ibs[u % 4].at[1]],
                                  adb[u % 2], gas[u % 2]).wait()

        pltpu.sync_copy(epk_hbm.at[gblk(0)], ib0)
        adjust_src(ib0)
        start_gathers(0)

        @pl.when(1 < nblk)
        def _():
            pltpu.async_copy(epk_hbm.at[gblk(1)], ib1, isem1)

        def quad(kk, _):
            for u in range(4):
                k = kk * 4 + u

                @pl.when(k + 1 < nblk)
                def _(u=u, k=k):
                    pltpu.make_async_copy(
                        epk_hbm.at[gblk(k + 1)],
                        ibs[(u + 1) % 4], isems[(u + 1) % 4]).wait()
                    adjust_src(ibs[(u + 1) % 4])
                    start_gathers(u + 1)

                @pl.when(jnp.logical_and(k >= 2, k <= nblk + 1))
                def _(u=u, k=k):
                    pltpu.make_async_copy(
                        msgb[u % 2], acc.at[ibs[(u + 2) % 4].at[1]],
                        sss[u % 2]).wait()

                @pl.when(k + 2 < nblk)
                def _(u=u, k=k):
                    pltpu.async_copy(epk_hbm.at[gblk(k + 2)],
                                     ibs[(u + 2) % 4], isems[(u + 2) % 4])

                @pl.when(k < nblk)
                def _(u=u, k=k):
                    wait_gathers(u)
                    compute_block(rowsb[u % 2], adb[u % 2], msgb[u % 2])
                    pltpu.async_copy(msgb[u % 2], acc.at[ibs[u % 4].at[1]],
                                     sss[u % 2], add=True)
            return 0

        lax.fori_loop(0, (nblk + 2 + 3) // 4, quad, 0)
        plsc.subcore_barrier()

        def drain(o_hbm):
            pltpu.sync_copy(acc.at[pl.ds(s * rpt, rpt)],
                            o_hbm.at[pl.ds(s * rpt, rpt)])
            if rem:
                @pl.when(s == NS - 1)
                def _():
                    pltpu.sync_copy(acc.at[pl.ds(NS * rpt, rem)],
                                    o_hbm.at[pl.ds(NS * rpt, rem)])

        @pl.when(c == 0)
        def _():
            drain(oa_hbm)

        @pl.when(c == 1)
        def _():
            drain(ob_hbm)

    kern = pl.kernel(
        body,
        out_type=[
            jax.ShapeDtypeStruct((n, AW), _f32),
            jax.ShapeDtypeStruct((n, AW), _f32),
        ],
        mesh=mesh,
        scratch_types=(
            [pltpu.VMEM_SHARED((n, AW), _f32)]
            + [pltpu.VMEM((2, EBLK), _i32) for _ in range(4)]
            + [pltpu.VMEM((EBLK, TW), _f32) for _ in range(2)]
            + [pltpu.VMEM((EBLK, 16), _f32) for _ in range(2)]
            + [pltpu.VMEM((EBLK, AW), _f32) for _ in range(2)]
            + [pltpu.SemaphoreType.DMA for _ in range(10)]
        ),
        compiler_params=pltpu.CompilerParams(use_tc_tiling_on_sc=False,
                                             needs_layout_passes=False),
    )
    return kern(tab, adt, epk)


def kernel(x, edge_index, Ws1, Wd1, as1, ad1, b1, Ws2, Wd2, as2, ad2, b2):
    n = x.shape[0]
    ei = edge_index.astype(_i32)
    src = ei[0]
    dst = ei[1]

    m1 = jnp.repeat(jnp.eye(HEADS, dtype=_f32), HID, axis=0)
    as8 = m1 * as1.reshape(-1)[:, None]
    ad8 = m1 * ad1.reshape(-1)[:, None]
    pad12 = jnp.zeros((HEADS * HID, 12), _f32)
    pad8 = jnp.zeros((HEADS * HID, 8), _f32)
    asa = jnp.concatenate([as8[:, :4], pad12], axis=1)
    asb = jnp.concatenate([as8[:, 4:], pad12], axis=1)
    ad16 = jnp.concatenate([ad8, pad8], axis=1)
    as2p = jnp.concatenate([as2.T, jnp.zeros((EMB, 15), _f32)], axis=1)
    ad2p = jnp.concatenate([ad2.T, jnp.zeros((EMB, 15), _f32)], axis=1)
    r8 = jnp.repeat(jnp.eye(HEADS, dtype=_f32), HID, axis=1)
    b1row = b1.reshape(1, -1)
    b2row = b2.reshape(1, -1)
    eb = src.shape[0] // EBLK
    epk = jnp.stack([src.reshape(eb, EBLK), dst.reshape(eb, EBLK)], axis=1)

    t1, ad1t = _phase_a(x, Ws1, Wd1, asa, asb, ad16)
    tab1 = t1.reshape(2 * n, TW)
    acc1a, acc1b = _sc_edge_layer(tab1, ad1t, epk, n,
                                  nheads=4, split_edges=False)
    t2, ad2t = _phase_c(acc1a, acc1b, b1row, Ws2, Wd2, as2p, ad2p, r8)
    acc2a, acc2b = _sc_edge_layer(t2, ad2t, epk, n,
                                  nheads=1, split_edges=True)
    return _phase_e(acc2a, acc2b, b2row)


# speedup vs baseline: 2.9162x; 1.3272x over previous
import functools

import jax
import jax.numpy as jnp
from jax import lax
from jax.experimental import pallas as pl
from jax.experimental.pallas import tpu as pltpu
from jax.experimental.pallas import tpu_sc as plsc

N_NODES = 10000
D_IN = 128
HEADS = 8
HID = 32
EMB = 128

NC = 2
NS = 16
LANES = 16
TW = 144
AW = 136
EBLK = 64

_f32 = jnp.float32
_i32 = jnp.int32


def _phase_a_body(x_ref, ws1_ref, wd1_ref, asa_ref, asb_ref, ad16_ref,
                  t1_ref, ad1t_ref):
    xb = x_ref[...]
    hs = jnp.dot(xb, ws1_ref[...], preferred_element_type=_f32)
    hd = jnp.dot(xb, wd1_ref[...], preferred_element_type=_f32)
    t1_ref[...] = jnp.concatenate(
        [hs[:, :128], jnp.dot(hs, asa_ref[...], preferred_element_type=_f32),
         hs[:, 128:], jnp.dot(hs, asb_ref[...], preferred_element_type=_f32)],
        axis=1)
    ad1t_ref[...] = jnp.dot(hd, ad16_ref[...], preferred_element_type=_f32)


def _phase_a(x, ws1, wd1, asa, asb, ad16):
    n = x.shape[0]
    blk = 1000
    grid = n // blk
    full = lambda shape: pl.BlockSpec(shape, lambda i: (0, 0))
    return pl.pallas_call(
        _phase_a_body,
        grid=(grid,),
        in_specs=[
            pl.BlockSpec((blk, D_IN), lambda i: (i, 0)),
            full((D_IN, HEADS * HID)),
            full((D_IN, HEADS * HID)),
            full((HEADS * HID, 16)),
            full((HEADS * HID, 16)),
            full((HEADS * HID, 16)),
        ],
        out_specs=[
            pl.BlockSpec((blk, 2 * TW), lambda i: (i, 0)),
            pl.BlockSpec((blk, 16), lambda i: (i, 0)),
        ],
        out_shape=[
            jax.ShapeDtypeStruct((n, 2 * TW), _f32),
            jax.ShapeDtypeStruct((n, 16), _f32),
        ],
    )(x, ws1, wd1, asa, asb, ad16)


def _phase_c_body(acca_ref, accb_ref, b1_ref, ws2_ref, wd2_ref, as2p_ref,
                  ad2p_ref, r8_ref, t2_ref, ad2t_ref):
    acca = acca_ref[...]
    accb = accb_ref[...]
    den8 = jnp.concatenate([acca[:, 128:132], accb[:, 128:132]], axis=1)
    rec8 = 1.0 / (den8 + 1e-16)
    scale = jnp.dot(rec8, r8_ref[...], preferred_element_type=_f32)
    num = jnp.concatenate([acca[:, :128], accb[:, :128]], axis=1)
    h1 = jnp.maximum(num * scale + b1_ref[...], 0.0)
    h2s = jnp.dot(h1, ws2_ref[...], preferred_element_type=_f32)
    hd2 = jnp.dot(h1, wd2_ref[...], preferred_element_type=_f32)
    t2_ref[...] = jnp.concatenate(
        [h2s, jnp.dot(h2s, as2p_ref[...], preferred_element_type=_f32)],
        axis=1)
    ad2t_ref[...] = jnp.dot(hd2, ad2p_ref[...], preferred_element_type=_f32)


def _phase_c(acca, accb, b1row, ws2, wd2, as2p, ad2p, r8):
    n = acca.shape[0]
    blk = 1000
    grid = n // blk
    full = lambda shape: pl.BlockSpec(shape, lambda i: (0, 0))
    d2 = HEADS * HID
    return pl.pallas_call(
        _phase_c_body,
        grid=(grid,),
        in_specs=[
            pl.BlockSpec((blk, AW), lambda i: (i, 0)),
            pl.BlockSpec((blk, AW), lambda i: (i, 0)),
            full((1, d2)),
            full((d2, EMB)),
            full((d2, EMB)),
            full((EMB, 16)),
            full((EMB, 16)),
            full((HEADS, d2)),
        ],
        out_specs=[
            pl.BlockSpec((blk, TW), lambda i: (i, 0)),
            pl.BlockSpec((blk, 16), lambda i: (i, 0)),
        ],
        out_shape=[
            jax.ShapeDtypeStruct((n, TW), _f32),
            jax.ShapeDtypeStruct((n, 16), _f32),
        ],
    )(acca, accb, b1row, ws2, wd2, as2p, ad2p, r8)


def _phase_e_body(acca_ref, accb_ref, b2_ref, out_ref):
    acca = acca_ref[...]
    accb = accb_ref[...]
    num = acca[:, :EMB] + accb[:, :EMB]
    den = acca[:, 128:129] + accb[:, 128:129]
    out_ref[...] = num / (den + 1e-16) + b2_ref[...]


def _phase_e(acca, accb, b2row):
    n = acca.shape[0]
    blk = 1000
    grid = n // blk
    return pl.pallas_call(
        _phase_e_body,
        grid=(grid,),
        in_specs=[
            pl.BlockSpec((blk, AW), lambda i: (i, 0)),
            pl.BlockSpec((blk, AW), lambda i: (i, 0)),
            pl.BlockSpec((1, EMB), lambda i: (0, 0)),
        ],
        out_specs=pl.BlockSpec((blk, EMB), lambda i: (i, 0)),
        out_shape=jax.ShapeDtypeStruct((n, EMB), _f32),
    )(acca, accb, b2row)


def _sc_edge_layer(tab, adt, epk, n, nheads, split_edges):
    e = epk.shape[0] * EBLK
    eb = e // EBLK
    share = eb // NC if split_edges else eb
    rpt = 8 * (n // (8 * NS))
    rem = n - NS * rpt

    mesh = plsc.VectorSubcoreMesh(core_axis_name="c", subcore_axis_name="s")

    def body(tab_hbm, adt_hbm, epk_hbm,
             oa_hbm, ob_hbm,
             acc, ib0, ib1, ib2, ib3, rows0, rows1, ad0, ad1m, msg0, msg1,
             isem0, isem1, isem2, isem3,
             grsem0, grsem1, gasem0, gasem1, ssem0, ssem1):
        ibs = [ib0, ib1, ib2, ib3]
        isems = [isem0, isem1, isem2, isem3]
        rowsb = [rows0, rows1]
        adb = [ad0, ad1m]
        msgb = [msg0, msg1]
        grs = [grsem0, grsem1]
        gas = [gasem0, gasem1]
        sss = [ssem0, ssem1]

        c = lax.axis_index("c")
        s = lax.axis_index("s")

        zcols = list(range(0, AW - 16, LANES)) + [AW - 16]

        def zfill(i, _):
            zv = jnp.zeros((LANES,), _f32)
            for col in zcols:
                msg0[i, pl.ds(col, LANES)] = zv
                msg1[i, pl.ds(col, LANES)] = zv
            return 0
        lax.fori_loop(0, EBLK, zfill, 0)

        nfull, tail = rpt // EBLK, rpt % EBLK

        def zacc(k, _):
            pltpu.sync_copy(msg0, acc.at[pl.ds(s * rpt + k * EBLK, EBLK)])
            return 0
        lax.fori_loop(0, nfull, zacc, 0)
        if tail:
            pltpu.sync_copy(msg0.at[pl.ds(0, tail)],
                            acc.at[pl.ds(s * rpt + nfull * EBLK, tail)])
        if rem:
            @pl.when(s == NS - 1)
            def _():
                pltpu.sync_copy(msg0.at[pl.ds(0, rem)],
                                acc.at[pl.ds(NS * rpt, rem)])
        plsc.subcore_barrier()

        iota16 = lax.iota(_i32, LANES)

        gbase = (c * share) if split_edges else 0
        nblk = (share - s + NS - 1) // NS
        ad_off = 0 if split_edges else c * nheads
        ch = 128 // nheads

        def gblk(k):
            return gbase + s + k * NS

        def adjust_src(ib):
            if not split_edges:
                coff = jnp.full((LANES,), c, _i32)
                for m in range(EBLK // LANES):
                    v = ib[0, pl.ds(m * LANES, LANES)]
                    ib[0, pl.ds(m * LANES, LANES)] = v + v + coff

        nv = 128 // LANES

        def compute_block(rows, adrows, msg):
            @functools.partial(plsc.parallel_loop, 0, EBLK // LANES,
                               unroll=2)
            def group(i):
                e16 = i * LANES + iota16
                exs = []
                for h in range(nheads):
                    asv = plsc.load_gather(
                        rows, [e16, jnp.full((LANES,), 128 + h, _i32)])
                    adv = plsc.load_gather(
                        adrows, [e16, jnp.full((LANES,), ad_off + h, _i32)])
                    al = asv + adv
                    ex = jnp.exp(jnp.maximum(al, 0.2 * al))
                    plsc.store_scatter(
                        msg, [e16, jnp.full((LANES,), 128 + h, _i32)], ex)
                    exs.append(ex)
                for j in range(LANES):
                    ei = i * LANES + j
                    loads = [rows[ei, pl.ds(v * LANES, LANES)]
                             for v in range(nv)]
                    vals = []
                    for h in range(nheads):
                        exv = jnp.full((LANES,), exs[h][j], _f32)
                        for q in range(ch // LANES):
                            vals.append(loads[h * (ch // LANES) + q] * exv)
                    for v in range(nv):
                        msg[ei, pl.ds(v * LANES, LANES)] = vals[v]

        def start_gathers(u):
            pltpu.async_copy(tab_hbm.at[ibs[u % 4].at[0]],
                             rowsb[u % 2], grs[u % 2])
            pltpu.async_copy(adt_hbm.at[ibs[u % 4].at[1]],
                             adb[u % 2], gas[u % 2])

        def wait_gathers(u):
            pltpu.make_async_copy(tab_hbm.at[ibs[u % 4].at[0]],
                                  rowsb[u % 2], grs[u % 2]).wait()
            pltpu.make_async_copy(adt_hbm.at[ibs[u % 4].at[1]],
                                  adb[u % 2], gas[u % 2]).wait()

        pltpu.sync_copy(epk_hbm.at[gblk(0)], ib0)
        adjust_src(ib0)
        start_gathers(0)

        @pl.when(1 < nblk)
        def _():
            pltpu.async_copy(epk_hbm.at[gblk(1)], ib1, isem1)

        def quad(kk, _):
            for u in range(4):
                k = kk * 4 + u

                @pl.when(k + 1 < nblk)
                def _(u=u, k=k):
                    pltpu.make_async_copy(
                        epk_hbm.at[gblk(k + 1)],
                        ibs[(u + 1) % 4], isems[(u + 1) % 4]).wait()
                    adjust_src(ibs[(u + 1) % 4])
                    start_gathers(u + 1)

                @pl.when(jnp.logical_and(k >= 2, k <= nblk + 1))
                def _(u=u, k=k):
                    pltpu.make_async_copy(
                        msgb[u % 2], acc.at[ibs[(u + 2) % 4].at[1]],
                        sss[u % 2]).wait()

                @pl.when(k + 2 < nblk)
                def _(u=u, k=k):
                    pltpu.async_copy(epk_hbm.at[gblk(k + 2)],
                                     ibs[(u + 2) % 4], isems[(u + 2) % 4])

                @pl.when(k < nblk)
                def _(u=u, k=k):
                    wait_gathers(u)
                    compute_block(rowsb[u % 2], adb[u % 2], msgb[u % 2])
                    pltpu.async_copy(msgb[u % 2], acc.at[ibs[u % 4].at[1]],
                                     sss[u % 2], add=True)
            return 0

        lax.fori_loop(0, (nblk + 2 + 3) // 4, quad, 0)
        plsc.subcore_barrier()

        def drain(o_hbm):
            pltpu.sync_copy(acc.at[pl.ds(s * rpt, rpt)],
                            o_hbm.at[pl.ds(s * rpt, rpt)])
            if rem:
                @pl.when(s == NS - 1)
                def _():
                    pltpu.sync_copy(acc.at[pl.ds(NS * rpt, rem)],
                                    o_hbm.at[pl.ds(NS * rpt, rem)])

        @pl.when(c == 0)
        def _():
            drain(oa_hbm)

        @pl.when(c == 1)
        def _():
            drain(ob_hbm)

    kern = pl.kernel(
        body,
        out_type=[
            jax.ShapeDtypeStruct((n, AW), _f32),
            jax.ShapeDtypeStruct((n, AW), _f32),
        ],
        mesh=mesh,
        scratch_types=(
            [pltpu.VMEM_SHARED((n, AW), _f32)]
            + [pltpu.VMEM((2, EBLK), _i32) for _ in range(4)]
            + [pltpu.VMEM((EBLK, TW), _f32) for _ in range(2)]
            + [pltpu.VMEM((EBLK, 16), _f32) for _ in range(2)]
            + [pltpu.VMEM((EBLK, AW), _f32) for _ in range(2)]
            + [pltpu.SemaphoreType.DMA for _ in range(10)]
        ),
        compiler_params=pltpu.CompilerParams(use_tc_tiling_on_sc=False,
                                             needs_layout_passes=False),
    )
    return kern(tab, adt, epk)


def kernel(x, edge_index, Ws1, Wd1, as1, ad1, b1, Ws2, Wd2, as2, ad2, b2):
    n = x.shape[0]
    ei = edge_index.astype(_i32)
    src = ei[0]
    dst = ei[1]

    m1 = jnp.repeat(jnp.eye(HEADS, dtype=_f32), HID, axis=0)
    as8 = m1 * as1.reshape(-1)[:, None]
    ad8 = m1 * ad1.reshape(-1)[:, None]
    pad12 = jnp.zeros((HEADS * HID, 12), _f32)
    pad8 = jnp.zeros((HEADS * HID, 8), _f32)
    asa = jnp.concatenate([as8[:, :4], pad12], axis=1)
    asb = jnp.concatenate([as8[:, 4:], pad12], axis=1)
    ad16 = jnp.concatenate([ad8, pad8], axis=1)
    as2p = jnp.concatenate([as2.T, jnp.zeros((EMB, 15), _f32)], axis=1)
    ad2p = jnp.concatenate([ad2.T, jnp.zeros((EMB, 15), _f32)], axis=1)
    r8 = jnp.repeat(jnp.eye(HEADS, dtype=_f32), HID, axis=1)
    b1row = b1.reshape(1, -1)
    b2row = b2.reshape(1, -1)
    eb = src.shape[0] // EBLK
    epk = jnp.stack([src.reshape(eb, EBLK), dst.reshape(eb, EBLK)], axis=1)

    t1, ad1t = _phase_a(x, Ws1, Wd1, asa, asb, ad16)
    tab1 = t1.reshape(2 * n, TW)
    acc1a, acc1b = _sc_edge_layer(tab1, ad1t, epk, n,
                                  nheads=4, split_edges=False)
    t2, ad2t = _phase_c(acc1a, acc1b, b1row, Ws2, Wd2, as2p, ad2p, r8)
    acc2a, acc2b = _sc_edge_layer(t2, ad2t, epk, n,
                                  nheads=1, split_edges=True)
    return _phase_e(acc2a, acc2b, b2row)
